# Initial kernel scaffold; baseline (speedup 1.0000x reference)
import jax
import jax.numpy as jnp
from jax.experimental import pallas as pl


def kernel(user_id, movie_title, user_table, movie_table):
    raise NotImplementedError("write your pallas kernel here")


# speedup vs baseline: 2.5169x; 2.5169x over previous
import functools

import jax
import jax.numpy as jnp
from jax import lax
from jax.experimental import pallas as pl
from jax.experimental.pallas import tpu as pltpu
from jax.experimental.pallas import tpu_sc as plsc

D = 64
N = 16384 * 20
NW = 32
PER_W = N // NW
CHUNK = 128
NCHUNK = PER_W // CHUNK

_mesh = plsc.VectorSubcoreMesh(core_axis_name="c", subcore_axis_name="s")


@functools.partial(
    pl.kernel,
    out_type=jax.ShapeDtypeStruct((N,), jnp.float32),
    mesh=_mesh,
    compiler_params=pltpu.CompilerParams(
        needs_layout_passes=False, use_tc_tiling_on_sc=False
    ),
    scratch_types=[
        pltpu.VMEM((NCHUNK, CHUNK), jnp.int32),
        pltpu.VMEM((NCHUNK, CHUNK), jnp.int32),
        pltpu.VMEM((PER_W,), jnp.float32),
        pltpu.VMEM((CHUNK, D), jnp.float32),
        pltpu.VMEM((CHUNK, D), jnp.float32),
        pltpu.VMEM((CHUNK, D), jnp.float32),
        pltpu.VMEM((CHUNK, D), jnp.float32),
        pltpu.SemaphoreType.DMA,
        pltpu.SemaphoreType.DMA,
        pltpu.SemaphoreType.DMA,
        pltpu.SemaphoreType.DMA,
    ],
)
def _sc_pair_dot(uid_hbm, mid_hbm, utab_hbm, mtab_hbm, out_hbm,
                 uidx, midx, outv, ru0, ru1, rm0, rm1, su0, su1, sm0, sm1):
    wid = lax.axis_index("s") * 2 + lax.axis_index("c")
    row0 = wid * NCHUNK

    pltpu.sync_copy(uid_hbm.at[pl.ds(row0, NCHUNK), :], uidx)
    pltpu.sync_copy(mid_hbm.at[pl.ds(row0, NCHUNK), :], midx)

    ru = (ru0, ru1)
    rm = (rm0, rm1)
    su = (su0, su1)
    sm = (sm0, sm1)

    def issue(c, b):
        pltpu.async_copy(utab_hbm.at[uidx.at[c]], ru[b], su[b])
        pltpu.async_copy(mtab_hbm.at[midx.at[c]], rm[b], sm[b])

    def wait(c, b):
        pltpu.make_async_copy(utab_hbm.at[uidx.at[c]], ru[b], su[b]).wait()
        pltpu.make_async_copy(mtab_hbm.at[midx.at[c]], rm[b], sm[b]).wait()

    issue(0, 0)
    issue(1, 1)

    lane = lax.iota(jnp.int32, 16)

    @pl.loop(0, NCHUNK, step=2)
    def _chunks(c):
        for b in range(2):
            cc = c + b
            wait(cc, b)

            base = cc * CHUNK

            @pl.loop(0, CHUNK // 16)
            def _groups(g):
                vec = jnp.zeros((16,), jnp.float32)
                for k in range(16):
                    j = g * 16 + k
                    a = ru[b][j, pl.ds(0, 16)] * rm[b][j, pl.ds(0, 16)]
                    a = a + ru[b][j, pl.ds(16, 16)] * rm[b][j, pl.ds(16, 16)]
                    a = a + ru[b][j, pl.ds(32, 16)] * rm[b][j, pl.ds(32, 16)]
                    a = a + ru[b][j, pl.ds(48, 16)] * rm[b][j, pl.ds(48, 16)]
                    vec = jnp.where(lane == k, jnp.sum(a), vec)
                off = pl.multiple_of(base + g * 16, 16)
                outv[pl.ds(off, 16)] = vec

            @pl.when(cc + 2 < NCHUNK)
            def _prefetch():
                issue(cc + 2, b)

    pltpu.sync_copy(outv, out_hbm.at[pl.ds(wid * PER_W, PER_W)])


def kernel(user_id, movie_title, user_table, movie_table):
    uid = user_id.reshape(N // CHUNK, CHUNK)
    mid = movie_title.reshape(N // CHUNK, CHUNK)
    out = _sc_pair_dot(uid, mid, user_table, movie_table)
    return out.reshape(user_id.shape)
